# Initial kernel scaffold; baseline (speedup 1.0000x reference)
import jax
import jax.numpy as jnp
from jax.experimental import pallas as pl


def kernel(atom_fea, nbr_fea, nbr_fea_idx, crystal_atom_idx, atom_type, nbr_type, nbr_dist, pair_type, global_fea, pool_atom_idx, emb_W, emb_b, convW, convb, bn1_g, bn1_b, bn2_g, bn2_b, fc_W, fc_b, out_W, out_b):
    raise NotImplementedError("write your pallas kernel here")


# speedup vs baseline: 1.7721x; 1.7721x over previous
import functools

import jax
import jax.numpy as jnp
from jax import lax
from jax.experimental import pallas as pl
from jax.experimental.pallas import tpu as pltpu
from jax.experimental.pallas import tpu_sc as plsc

N = 10000
M = 32
ORIG = 92
NBR = 16
AF = 64
HF = 128
NCONV = 3
NCRY = 100
PER = 100
E = N * M

NW = 32
GB = 128
GK = 79
E_PAD = NW * GK * GB

AT = 200
ET = AT * M
GRID_E = N // AT

_EPS = 1e-5


def _softplus(x):
    return jnp.maximum(x, 0.0) + jnp.log1p(jnp.exp(-jnp.abs(x)))


def _make_sc_gather():
    mesh = plsc.VectorSubcoreMesh(
        core_axis_name="c", subcore_axis_name="s", num_cores=2, num_subcores=16
    )

    @functools.partial(
        pl.kernel,
        out_type=jax.ShapeDtypeStruct((E_PAD, AF), jnp.float32),
        mesh=mesh,
        scratch_types=[
            pltpu.VMEM((GK, GB), jnp.int32),
            pltpu.VMEM((GB, AF), jnp.float32),
            pltpu.SemaphoreType.DMA,
        ],
        compiler_params=pltpu.CompilerParams(use_tc_tiling_on_sc=False),
    )
    def gather_k(tbl_hbm, idx_hbm, out_hbm, idx_v, rows_v, sem):
        wid = lax.axis_index("s") * 2 + lax.axis_index("c")
        base = wid * (GK * GB)
        pltpu.sync_copy(idx_hbm.at[wid], idx_v)

        def body(j, carry):
            pltpu.async_copy(tbl_hbm.at[idx_v.at[j]], rows_v, sem).wait()
            pltpu.sync_copy(rows_v, out_hbm.at[pl.ds(base + j * GB, GB)])
            return carry

        lax.fori_loop(0, GK, body, 0)

    return gather_k


_SC_GATHER_CACHE = []


def _sc_gather(tbl, idx3):
    if not _SC_GATHER_CACHE:
        _SC_GATHER_CACHE.append(_make_sc_gather())
    return _SC_GATHER_CACHE[0](tbl, idx3)


def _embed_body(af_ref, ew_ref, eb_ref, ws_ref, cb_ref, x_ref, p_ref):
    x = jnp.dot(af_ref[...], ew_ref[...], preferred_element_type=jnp.float32)
    x = x + eb_ref[...]
    x_ref[...] = x
    p_ref[...] = jnp.dot(x, ws_ref[...], preferred_element_type=jnp.float32) + cb_ref[...]


def _embed(atom_fea, emb_W, emb_b, w_self, conv_b):
    bt = 1000
    return pl.pallas_call(
        _embed_body,
        grid=(N // bt,),
        in_specs=[
            pl.BlockSpec((bt, ORIG), lambda i: (i, 0)),
            pl.BlockSpec((ORIG, AF), lambda i: (0, 0)),
            pl.BlockSpec((1, AF), lambda i: (0, 0)),
            pl.BlockSpec((AF, 2 * AF), lambda i: (0, 0)),
            pl.BlockSpec((1, 2 * AF), lambda i: (0, 0)),
        ],
        out_specs=[
            pl.BlockSpec((bt, AF), lambda i: (i, 0)),
            pl.BlockSpec((bt, 2 * AF), lambda i: (i, 0)),
        ],
        out_shape=[
            jax.ShapeDtypeStruct((N, AF), jnp.float32),
            jax.ShapeDtypeStruct((N, 2 * AF), jnp.float32),
        ],
    )(atom_fea, emb_W, emb_b.reshape(1, AF), w_self, conv_b.reshape(1, 2 * AF))


def _gated_tile(gx_ref, nf_ref, p_ref, wn_ref, we_ref):
    g = jnp.dot(gx_ref[...], wn_ref[...], preferred_element_type=jnp.float32)
    g = g + jnp.dot(
        nf_ref[...].reshape(ET, NBR), we_ref[...], preferred_element_type=jnp.float32
    )
    p = p_ref[...]
    g = g + jnp.broadcast_to(p[:, None, :], (AT, M, 2 * AF)).reshape(ET, 2 * AF)
    return g


def _stats_body(gx_ref, nf_ref, p_ref, wn_ref, we_ref, sum_ref, sq_ref):
    i = pl.program_id(0)
    g = _gated_tile(gx_ref, nf_ref, p_ref, wn_ref, we_ref)

    @pl.when(i == 0)
    def _():
        sum_ref[...] = jnp.zeros_like(sum_ref)
        sq_ref[...] = jnp.zeros_like(sq_ref)

    sum_ref[...] += jnp.sum(g, axis=0, keepdims=True)
    sq_ref[...] += jnp.sum(g * g, axis=0, keepdims=True)


def _stats(gx, nbr_fea, p, w_nbr, w_edge):
    return pl.pallas_call(
        _stats_body,
        grid=(GRID_E,),
        in_specs=[
            pl.BlockSpec((ET, AF), lambda i: (i, 0)),
            pl.BlockSpec((AT, M, NBR), lambda i: (i, 0, 0)),
            pl.BlockSpec((AT, 2 * AF), lambda i: (i, 0)),
            pl.BlockSpec((AF, 2 * AF), lambda i: (0, 0)),
            pl.BlockSpec((NBR, 2 * AF), lambda i: (0, 0)),
        ],
        out_specs=[
            pl.BlockSpec((1, 2 * AF), lambda i: (0, 0)),
            pl.BlockSpec((1, 2 * AF), lambda i: (0, 0)),
        ],
        out_shape=[
            jax.ShapeDtypeStruct((1, 2 * AF), jnp.float32),
            jax.ShapeDtypeStruct((1, 2 * AF), jnp.float32),
        ],
    )(gx, nbr_fea, p, w_nbr, w_edge)


def _apply_body(gx_ref, nf_ref, p_ref, wn_ref, we_ref, sc_ref, sh_ref,
                ns_ref, s2_ref, q2_ref):
    i = pl.program_id(0)
    g = _gated_tile(gx_ref, nf_ref, p_ref, wn_ref, we_ref)
    g = g * sc_ref[...] + sh_ref[...]
    filt = jax.nn.sigmoid(g[:, :AF])
    core = _softplus(g[:, AF:])
    ns = jnp.sum((filt * core).reshape(AT, M, AF), axis=1)
    ns_ref[...] = ns

    @pl.when(i == 0)
    def _():
        s2_ref[...] = jnp.zeros_like(s2_ref)
        q2_ref[...] = jnp.zeros_like(q2_ref)

    s2_ref[...] += jnp.sum(ns, axis=0, keepdims=True)
    q2_ref[...] += jnp.sum(ns * ns, axis=0, keepdims=True)


def _apply(gx, nbr_fea, p, w_nbr, w_edge, scale1, shift1):
    return pl.pallas_call(
        _apply_body,
        grid=(GRID_E,),
        in_specs=[
            pl.BlockSpec((ET, AF), lambda i: (i, 0)),
            pl.BlockSpec((AT, M, NBR), lambda i: (i, 0, 0)),
            pl.BlockSpec((AT, 2 * AF), lambda i: (i, 0)),
            pl.BlockSpec((AF, 2 * AF), lambda i: (0, 0)),
            pl.BlockSpec((NBR, 2 * AF), lambda i: (0, 0)),
            pl.BlockSpec((1, 2 * AF), lambda i: (0, 0)),
            pl.BlockSpec((1, 2 * AF), lambda i: (0, 0)),
        ],
        out_specs=[
            pl.BlockSpec((AT, AF), lambda i: (i, 0)),
            pl.BlockSpec((1, AF), lambda i: (0, 0)),
            pl.BlockSpec((1, AF), lambda i: (0, 0)),
        ],
        out_shape=[
            jax.ShapeDtypeStruct((N, AF), jnp.float32),
            jax.ShapeDtypeStruct((1, AF), jnp.float32),
            jax.ShapeDtypeStruct((1, AF), jnp.float32),
        ],
    )(gx, nbr_fea, p, w_nbr, w_edge, scale1, shift1)


def _update_body_p(x_ref, ns_ref, sc_ref, sh_ref, ws_ref, cb_ref, xo_ref, p_ref):
    xn = _softplus(x_ref[...] + ns_ref[...] * sc_ref[...] + sh_ref[...])
    xo_ref[...] = xn
    p_ref[...] = jnp.dot(xn, ws_ref[...], preferred_element_type=jnp.float32) + cb_ref[...]


def _update_body(x_ref, ns_ref, sc_ref, sh_ref, xo_ref):
    xo_ref[...] = _softplus(x_ref[...] + ns_ref[...] * sc_ref[...] + sh_ref[...])


def _update(x, ns, scale2, shift2, w_self=None, conv_b=None):
    bt = 1000
    if w_self is None:
        return pl.pallas_call(
            _update_body,
            grid=(N // bt,),
            in_specs=[
                pl.BlockSpec((bt, AF), lambda i: (i, 0)),
                pl.BlockSpec((bt, AF), lambda i: (i, 0)),
                pl.BlockSpec((1, AF), lambda i: (0, 0)),
                pl.BlockSpec((1, AF), lambda i: (0, 0)),
            ],
            out_specs=pl.BlockSpec((bt, AF), lambda i: (i, 0)),
            out_shape=jax.ShapeDtypeStruct((N, AF), jnp.float32),
        )(x, ns, scale2, shift2)
    return pl.pallas_call(
        _update_body_p,
        grid=(N // bt,),
        in_specs=[
            pl.BlockSpec((bt, AF), lambda i: (i, 0)),
            pl.BlockSpec((bt, AF), lambda i: (i, 0)),
            pl.BlockSpec((1, AF), lambda i: (0, 0)),
            pl.BlockSpec((1, AF), lambda i: (0, 0)),
            pl.BlockSpec((AF, 2 * AF), lambda i: (0, 0)),
            pl.BlockSpec((1, 2 * AF), lambda i: (0, 0)),
        ],
        out_specs=[
            pl.BlockSpec((bt, AF), lambda i: (i, 0)),
            pl.BlockSpec((bt, 2 * AF), lambda i: (i, 0)),
        ],
        out_shape=[
            jax.ShapeDtypeStruct((N, AF), jnp.float32),
            jax.ShapeDtypeStruct((N, 2 * AF), jnp.float32),
        ],
    )(x, ns, scale2, shift2, w_self, conv_b.reshape(1, 2 * AF))


def _head_body(x_ref, fw_ref, fb_ref, ow_ref, ob_ref, out_ref):
    crys = jnp.mean(x_ref[...].reshape(NCRY, PER, AF), axis=1)
    h = _softplus(
        jnp.dot(_softplus(crys), fw_ref[...], preferred_element_type=jnp.float32)
        + fb_ref[...]
    )
    out_ref[...] = jnp.dot(h, ow_ref[...], preferred_element_type=jnp.float32) + ob_ref[...]


def _head(x, fc_W, fc_b, out_W, out_b):
    return pl.pallas_call(
        _head_body,
        in_specs=[
            pl.BlockSpec((N, AF), lambda: (0, 0)),
            pl.BlockSpec((AF, HF), lambda: (0, 0)),
            pl.BlockSpec((1, HF), lambda: (0, 0)),
            pl.BlockSpec((HF, 1), lambda: (0, 0)),
            pl.BlockSpec((1, 1), lambda: (0, 0)),
        ],
        out_specs=pl.BlockSpec((NCRY, 1), lambda: (0, 0)),
        out_shape=jax.ShapeDtypeStruct((NCRY, 1), jnp.float32),
    )(x, fc_W, fc_b.reshape(1, HF), out_W, out_b.reshape(1, 1))


def kernel(atom_fea, nbr_fea, nbr_fea_idx, crystal_atom_idx, atom_type,
           nbr_type, nbr_dist, pair_type, global_fea, pool_atom_idx,
           emb_W, emb_b, convW, convb, bn1_g, bn1_b, bn2_g, bn2_b,
           fc_W, fc_b, out_W, out_b):
    flat_idx = nbr_fea_idx.astype(jnp.int32).reshape(-1)
    idx3 = jnp.concatenate(
        [flat_idx, jnp.zeros((E_PAD - E,), jnp.int32)]
    ).reshape(NW, GK, GB)

    w_self = convW[:, :AF, :]
    w_nbr = convW[:, AF:2 * AF, :]
    w_edge = convW[:, 2 * AF:, :]

    x, p = _embed(atom_fea, emb_W, emb_b, w_self[0], convb[0])

    for i in range(NCONV):
        gx = _sc_gather(x, idx3)
        ssum, ssq = _stats(gx, nbr_fea, p, w_nbr[i], w_edge[i])
        mu = ssum / E
        var = ssq / E - mu * mu
        scale1 = (bn1_g[i] / jnp.sqrt(var + _EPS)).reshape(1, 2 * AF)
        shift1 = (bn1_b[i] - mu * scale1).reshape(1, 2 * AF)
        ns, s2, q2 = _apply(gx, nbr_fea, p, w_nbr[i], w_edge[i], scale1, shift1)
        mu2 = s2 / N
        var2 = q2 / N - mu2 * mu2
        scale2 = (bn2_g[i] / jnp.sqrt(var2 + _EPS)).reshape(1, AF)
        shift2 = (bn2_b[i] - mu2 * scale2).reshape(1, AF)
        if i + 1 < NCONV:
            x, p = _update(x, ns, scale2, shift2, w_self[i + 1], convb[i + 1])
        else:
            x = _update(x, ns, scale2, shift2)

    return _head(x, fc_W, fc_b, out_W, out_b)
